# baseline (speedup 1.0000x reference)
import functools

import jax
import jax.numpy as jnp
from jax import lax
from jax.experimental import pallas as pl
from jax.experimental.pallas import tpu as pltpu
from jax.experimental.pallas import tpu_sc as plsc

VOCAB = 1000000
EMB = 64
B = 4096
S = 200
NC = 2
NS = 16
NW = NC * NS
BT = B // NW
NBT = B // 128
NB = 4
NG = S // NB
SCALE = 1.0 / 64.0
L = 16


@functools.partial(
    pl.kernel,
    mesh=plsc.VectorSubcoreMesh(core_axis_name="c", subcore_axis_name="s"),
    out_type=jax.ShapeDtypeStruct((S, EMB // 8, NBT, 8, 128), jnp.float32),
    compiler_params=pltpu.CompilerParams(
        use_tc_tiling_on_sc=False, needs_layout_passes=False),
    scratch_types=(
        [pltpu.VMEM((BT,), jnp.int32) for _ in range(NB)]
        + [pltpu.VMEM((BT,), jnp.int32) for _ in range(NB)]
        + [pltpu.VMEM((BT, 2 * EMB), jnp.float32) for _ in range(NB)]
        + [pltpu.VMEM((EMB // 8, 8, BT), jnp.float32) for _ in range(NB)]
        + [pltpu.SemaphoreType.DMA for _ in range(2 * NB)]
    ),
)
def _emb_lookup(idx_hbm, table_hbm, out_hbm,
                pv0, pv1, pv2, pv3, hv0, hv1, hv2, hv3,
                b0, b1, b2, b3, t0, t1, t2, t3,
                g0, g1, g2, g3, o0, o1, o2, o3):
    pvs = (pv0, pv1, pv2, pv3)
    hvs = (hv0, hv1, hv2, hv3)
    bufs = (b0, b1, b2, b3)
    ots = (t0, t1, t2, t3)
    gsems = (g0, g1, g2, g3)
    osems = (o0, o1, o2, o3)
    wid = lax.axis_index("s") * NC + lax.axis_index("c")
    boff = wid * BT
    lanes = lax.iota(jnp.int32, L)

    def out_slab(s):
        return out_hbm.at[s, pl.ds(0, EMB // 8), wid, pl.ds(0, 8), pl.ds(0, 128)]

    def prep_and_gather(s, k):
        pltpu.sync_copy(idx_hbm.at[pl.ds(s * B + boff, BT)], pvs[k])
        for g in range(BT // L):
            sl = pl.ds(g * L, L)
            v = pvs[k][sl]
            hvs[k][sl] = lax.shift_left(
                lax.bitwise_and(v, jnp.int32(1)), jnp.int32(6))
            pvs[k][sl] = lax.shift_right_logical(v, jnp.int32(1))
        pltpu.async_copy(table_hbm.at[pvs[k]], bufs[k], gsems[k])

    def wait_gather(k):
        pltpu.make_async_copy(table_hbm.at[pvs[k]], bufs[k], gsems[k]).wait()

    def wait_store(k):
        pltpu.make_async_copy(ots[k], out_slab(0), osems[k]).wait()

    for k in range(NB):
        prep_and_gather(k, k)

    def group_body(g, carry):
        for k in range(NB):
            s = g * NB + k
            wait_gather(k)

            def block_body(gg, c, k=k):
                rowv = gg * L + lanes
                colb = hvs[k][pl.ds(gg * L, L)]
                for j in range(EMB):
                    vec = plsc.load_gather(bufs[k], [rowv, colb + j])
                    ots[k][j // 8, j % 8, pl.ds(gg * L, L)] = vec * SCALE
                return c

            lax.fori_loop(0, BT // L, block_body, 0)
            pltpu.async_copy(ots[k], out_slab(s), osems[k])

        @pl.when(g + 1 < NG)
        def _prefetch():
            for k in range(NB):
                wait_store(k)
                prep_and_gather((g + 1) * NB + k, k)

        return carry

    lax.fori_loop(0, NG, group_body, 0)
    for k in range(NB):
        wait_store(k)


def kernel(inp, table):
    idx_flat = jnp.swapaxes(inp, 0, 1).reshape(S * B)
    table2 = table.reshape(VOCAB // 2, 2 * EMB)
    out = _emb_lookup(idx_flat, table2)
    return jnp.transpose(out, (2, 4, 0, 1, 3)).reshape(B, S, EMB)


# speedup vs baseline: 1.0123x; 1.0123x over previous
import functools

import jax
import jax.numpy as jnp
from jax import lax
from jax.experimental import pallas as pl
from jax.experimental.pallas import tpu as pltpu
from jax.experimental.pallas import tpu_sc as plsc

VOCAB = 1000000
EMB = 64
B = 4096
S = 200
NC = 2
NS = 16
NW = NC * NS
BT = B // NW
NBT = B // 128
NB = 4
NG = S // NB
SCALE = 1.0 / 64.0
L = 16


@functools.partial(
    pl.kernel,
    mesh=plsc.VectorSubcoreMesh(core_axis_name="c", subcore_axis_name="s"),
    out_type=jax.ShapeDtypeStruct((S, EMB // 8, NBT, 8, 128), jnp.float32),
    compiler_params=pltpu.CompilerParams(
        use_tc_tiling_on_sc=False, needs_layout_passes=False),
    scratch_types=(
        [pltpu.VMEM((BT,), jnp.int32) for _ in range(NB)]
        + [pltpu.VMEM((BT, EMB), jnp.float32) for _ in range(NB)]
        + [pltpu.VMEM((EMB // 8, 8, BT), jnp.float32) for _ in range(NB)]
        + [pltpu.SemaphoreType.DMA for _ in range(2 * NB)]
    ),
)
def _emb_lookup(idx_hbm, table_hbm, out_hbm,
                pv0, pv1, pv2, pv3,
                b0, b1, b2, b3, t0, t1, t2, t3,
                g0, g1, g2, g3, o0, o1, o2, o3):
    pvs = (pv0, pv1, pv2, pv3)
    bufs = (b0, b1, b2, b3)
    ots = (t0, t1, t2, t3)
    gsems = (g0, g1, g2, g3)
    osems = (o0, o1, o2, o3)
    wid = lax.axis_index("s") * NC + lax.axis_index("c")
    boff = wid * BT
    lanes = lax.iota(jnp.int32, L)
    zrow = lanes * 0

    def out_slab(s):
        return out_hbm.at[s, pl.ds(0, EMB // 8), wid, pl.ds(0, 8), pl.ds(0, 128)]

    def prep_and_gather(s, k):
        pltpu.sync_copy(idx_hbm.at[pl.ds(s * B + boff, BT)], pvs[k])
        pltpu.async_copy(table_hbm.at[pvs[k]], bufs[k], gsems[k])

    def wait_gather(k):
        pltpu.make_async_copy(table_hbm.at[pvs[k]], bufs[k], gsems[k]).wait()

    def wait_store(k):
        pltpu.make_async_copy(ots[k], out_slab(0), osems[k]).wait()

    for k in range(NB):
        prep_and_gather(k, k)

    def group_body(g, carry):
        for k in range(NB):
            s = g * NB + k
            wait_gather(k)

            def block_body(gg, c, k=k):
                rowv = gg * L + lanes
                for j in range(EMB):
                    vec = plsc.load_gather(bufs[k], [rowv, zrow + j])
                    ots[k][j // 8, j % 8, pl.ds(gg * L, L)] = vec * SCALE
                return c

            lax.fori_loop(0, BT // L, block_body, 0)
            pltpu.async_copy(ots[k], out_slab(s), osems[k])

        @pl.when(g + 1 < NG)
        def _prefetch():
            for k in range(NB):
                wait_store(k)
                prep_and_gather((g + 1) * NB + k, k)

        return carry

    lax.fori_loop(0, NG, group_body, 0)
    for k in range(NB):
        wait_store(k)


def kernel(inp, table):
    idx_flat = jnp.swapaxes(inp, 0, 1).reshape(S * B)
    out = _emb_lookup(idx_flat, table)
    return jnp.transpose(out, (2, 4, 0, 1, 3)).reshape(B, S, EMB)


# speedup vs baseline: 1.1371x; 1.1233x over previous
import functools

import jax
import jax.numpy as jnp
from jax import lax
from jax.experimental import pallas as pl
from jax.experimental.pallas import tpu as pltpu
from jax.experimental.pallas import tpu_sc as plsc

VOCAB = 1000000
EMB = 64
B = 4096
S = 200
NC = 2
NS = 16
NW = NC * NS
BT = B // NW
NBT = B // 128
NB = 4
NG = S // NB
SCALE = 1.0 / 64.0
L = 16


@functools.partial(
    pl.kernel,
    mesh=plsc.VectorSubcoreMesh(core_axis_name="c", subcore_axis_name="s"),
    out_type=jax.ShapeDtypeStruct((S, EMB // 8, NBT, 8, 128), jnp.float32),
    compiler_params=pltpu.CompilerParams(
        use_tc_tiling_on_sc=False, needs_layout_passes=False),
    scratch_types=(
        [pltpu.VMEM((BT,), jnp.int32) for _ in range(NB)]
        + [pltpu.VMEM((BT, EMB), jnp.float32) for _ in range(NB)]
        + [pltpu.VMEM((EMB // 8, 8, BT), jnp.float32) for _ in range(NB)]
        + [pltpu.SemaphoreType.DMA for _ in range(2 * NB)]
    ),
)
def _emb_lookup(idx_hbm, table_hbm, out_hbm,
                pv0, pv1, pv2, pv3,
                b0, b1, b2, b3, t0, t1, t2, t3,
                g0, g1, g2, g3, o0, o1, o2, o3):
    pvs = (pv0, pv1, pv2, pv3)
    bufs = (b0, b1, b2, b3)
    ots = (t0, t1, t2, t3)
    gsems = (g0, g1, g2, g3)
    osems = (o0, o1, o2, o3)
    wid = lax.axis_index("s") * NC + lax.axis_index("c")
    boff = wid * BT
    lanes = lax.iota(jnp.int32, L)
    zrow = lanes * 0
    ehi = [lax.shift_right_logical(t * L + lanes, jnp.int32(3))
           for t in range(EMB // L)]
    elo = [lax.bitwise_and(t * L + lanes, jnp.int32(7))
           for t in range(EMB // L)]

    def out_slab(s):
        return out_hbm.at[s, pl.ds(0, EMB // 8), wid, pl.ds(0, 8), pl.ds(0, 128)]

    def prep_and_gather(s, k):
        pltpu.sync_copy(idx_hbm.at[pl.ds(s * B + boff, BT)], pvs[k])
        pltpu.async_copy(table_hbm.at[pvs[k]], bufs[k], gsems[k])

    def wait_gather(k):
        pltpu.make_async_copy(table_hbm.at[pvs[k]], bufs[k], gsems[k]).wait()

    def wait_store(k):
        pltpu.make_async_copy(ots[k], out_slab(0), osems[k]).wait()

    for k in range(NB):
        prep_and_gather(k, k)

    def group_body(g, carry):
        for k in range(NB):
            s = g * NB + k
            wait_gather(k)

            def row_body(b, c, k=k):
                bs = zrow + b
                for t in range(EMB // L):
                    v = bufs[k][b, pl.ds(t * L, L)] * SCALE
                    plsc.store_scatter(ots[k], [ehi[t], elo[t], bs], v)
                return c

            lax.fori_loop(0, BT, row_body, 0, unroll=2)
            pltpu.async_copy(ots[k], out_slab(s), osems[k])

        @pl.when(g + 1 < NG)
        def _prefetch():
            for k in range(NB):
                wait_store(k)
                prep_and_gather((g + 1) * NB + k, k)

        return carry

    lax.fori_loop(0, NG, group_body, 0)
    for k in range(NB):
        wait_store(k)


def kernel(inp, table):
    idx_flat = jnp.swapaxes(inp, 0, 1).reshape(S * B)
    out = _emb_lookup(idx_flat, table)
    return jnp.transpose(out, (2, 4, 0, 1, 3)).reshape(B, S, EMB)


# speedup vs baseline: 1.1434x; 1.0055x over previous
import functools

import jax
import jax.numpy as jnp
from jax import lax
from jax.experimental import pallas as pl
from jax.experimental.pallas import tpu as pltpu
from jax.experimental.pallas import tpu_sc as plsc

VOCAB = 1000000
EMB = 64
B = 4096
S = 200
NC = 2
NS = 16
NW = NC * NS
BT = B // NW
NBT = B // 128
NB = 4
NG = S // NB
SCALE = 1.0 / 64.0
L = 16


@functools.partial(
    pl.kernel,
    mesh=plsc.VectorSubcoreMesh(core_axis_name="c", subcore_axis_name="s"),
    out_type=jax.ShapeDtypeStruct((S, EMB // 8, NBT, 8, 128), jnp.float32),
    compiler_params=pltpu.CompilerParams(
        use_tc_tiling_on_sc=False, needs_layout_passes=False),
    scratch_types=(
        [pltpu.VMEM((S, BT), jnp.int32)]
        + [pltpu.VMEM((BT, EMB), jnp.float32) for _ in range(NB)]
        + [pltpu.VMEM((EMB // 8, 8, BT), jnp.float32) for _ in range(NB)]
        + [pltpu.SemaphoreType.DMA for _ in range(2 * NB)]
    ),
)
def _emb_lookup(idx_hbm, table_hbm, out_hbm, idxv,
                b0, b1, b2, b3, t0, t1, t2, t3,
                g0, g1, g2, g3, o0, o1, o2, o3):
    bufs = (b0, b1, b2, b3)
    ots = (t0, t1, t2, t3)
    gsems = (g0, g1, g2, g3)
    osems = (o0, o1, o2, o3)
    wid = lax.axis_index("s") * NC + lax.axis_index("c")
    boff = wid * BT
    lanes = lax.iota(jnp.int32, L)
    zrow = lanes * 0
    ehi = [lax.shift_right_logical(t * L + lanes, jnp.int32(3))
           for t in range(EMB // L)]
    elo = [lax.bitwise_and(t * L + lanes, jnp.int32(7))
           for t in range(EMB // L)]

    pltpu.sync_copy(idx_hbm.at[pl.ds(0, S), pl.ds(boff, BT)], idxv)

    def out_slab(s):
        return out_hbm.at[s, pl.ds(0, EMB // 8), wid, pl.ds(0, 8), pl.ds(0, 128)]

    def start_gather(s, k):
        pltpu.async_copy(table_hbm.at[idxv.at[s]], bufs[k], gsems[k])

    def wait_gather(k):
        pltpu.make_async_copy(table_hbm.at[idxv.at[0]], bufs[k], gsems[k]).wait()

    def wait_store(k):
        pltpu.make_async_copy(ots[k], out_slab(0), osems[k]).wait()

    for k in range(NB):
        start_gather(k, k)

    def group_body(g, carry):
        for k in range(NB):
            s = g * NB + k
            wait_gather(k)

            def row_body(b, c, k=k):
                bs = zrow + b
                for t in range(EMB // L):
                    v = bufs[k][b, pl.ds(t * L, L)] * SCALE
                    plsc.store_scatter(ots[k], [ehi[t], elo[t], bs], v)
                return c

            lax.fori_loop(0, BT, row_body, 0, unroll=2)
            pltpu.async_copy(ots[k], out_slab(s), osems[k])

        @pl.when(g + 1 < NG)
        def _prefetch():
            for k in range(NB):
                wait_store(k)
                start_gather((g + 1) * NB + k, k)

        return carry

    lax.fori_loop(0, NG, group_body, 0)
    for k in range(NB):
        wait_store(k)


def kernel(inp, table):
    idx_t = jnp.swapaxes(inp, 0, 1)
    out = _emb_lookup(idx_t, table)
    return jnp.transpose(out, (2, 4, 0, 1, 3)).reshape(B, S, EMB)


# speedup vs baseline: 1.1462x; 1.0025x over previous
import functools

import jax
import jax.numpy as jnp
from jax import lax
from jax.experimental import pallas as pl
from jax.experimental.pallas import tpu as pltpu
from jax.experimental.pallas import tpu_sc as plsc

VOCAB = 1000000
EMB = 64
B = 4096
S = 200
NC = 2
NS = 16
NW = NC * NS
BT = B // NW
NBT = B // 128
NB = 4
NG = S // NB
SCALE = 1.0 / 64.0
L = 16


@functools.partial(
    pl.kernel,
    mesh=plsc.VectorSubcoreMesh(core_axis_name="c", subcore_axis_name="s"),
    out_type=jax.ShapeDtypeStruct((S, EMB // 8, NBT, 8, 128), jnp.float32),
    compiler_params=pltpu.CompilerParams(
        use_tc_tiling_on_sc=False, needs_layout_passes=False),
    scratch_types=(
        [pltpu.VMEM((S, BT), jnp.int32)]
        + [pltpu.VMEM((BT, EMB), jnp.float32) for _ in range(NB)]
        + [pltpu.VMEM((EMB // 8, 8, BT), jnp.float32) for _ in range(NB)]
        + [pltpu.SemaphoreType.DMA for _ in range(2 * NB)]
    ),
)
def _emb_lookup(idx_hbm, table_hbm, out_hbm, idxv,
                b0, b1, b2, b3, t0, t1, t2, t3,
                g0, g1, g2, g3, o0, o1, o2, o3):
    bufs = (b0, b1, b2, b3)
    ots = (t0, t1, t2, t3)
    gsems = (g0, g1, g2, g3)
    osems = (o0, o1, o2, o3)
    wid = lax.axis_index("s") * NC + lax.axis_index("c")
    boff = wid * BT
    lanes = lax.iota(jnp.int32, L)
    zrow = lanes * 0
    ehi = [lax.shift_right_logical(t * L + lanes, jnp.int32(3))
           for t in range(EMB // L)]
    elo = [lax.bitwise_and(t * L + lanes, jnp.int32(7))
           for t in range(EMB // L)]

    pltpu.sync_copy(idx_hbm.at[pl.ds(0, S), pl.ds(boff, BT)], idxv)

    def tile_dst(s, e8):
        return out_hbm.at[s, e8, wid, pl.ds(0, 8), pl.ds(0, 128)]

    def store_tiles(s, k):
        for e8 in range(EMB // 8):
            pltpu.async_copy(ots[k].at[e8], tile_dst(s, e8), osems[k])

    def wait_store(k):
        for e8 in range(EMB // 8):
            pltpu.make_async_copy(ots[k].at[e8], tile_dst(0, e8),
                                  osems[k]).wait()

    def start_gather(s, k):
        pltpu.async_copy(table_hbm.at[idxv.at[s]], bufs[k], gsems[k])

    def wait_gather(k):
        pltpu.make_async_copy(table_hbm.at[idxv.at[0]], bufs[k], gsems[k]).wait()

    for k in range(NB):
        start_gather(k, k)

    def group_body(g, carry):
        for k in range(NB):
            s = g * NB + k
            wait_gather(k)

            def row_body(b, c, k=k):
                bs = zrow + b
                for t in range(EMB // L):
                    v = bufs[k][b, pl.ds(t * L, L)] * SCALE
                    plsc.store_scatter(ots[k], [ehi[t], elo[t], bs], v)
                return c

            lax.fori_loop(0, BT, row_body, 0, unroll=2)
            store_tiles(s, k)

        @pl.when(g + 1 < NG)
        def _prefetch():
            for k in range(NB):
                wait_store(k)
                start_gather((g + 1) * NB + k, k)

        return carry

    lax.fori_loop(0, NG, group_body, 0)
    for k in range(NB):
        wait_store(k)


def kernel(inp, table):
    idx_t = jnp.swapaxes(inp, 0, 1)
    out = _emb_lookup(idx_t, table)
    return jnp.transpose(out, (2, 4, 0, 1, 3)).reshape(B, S, EMB)


# speedup vs baseline: 1.2798x; 1.1166x over previous
import functools

import jax
import jax.numpy as jnp
from jax import lax
from jax.experimental import pallas as pl
from jax.experimental.pallas import tpu as pltpu
from jax.experimental.pallas import tpu_sc as plsc

VOCAB = 1000000
EMB = 64
B = 4096
S = 200
NC = 2
NS = 16
NW = NC * NS
BT = B // NW
NBT = B // 128
NB = 4
NG = S // NB
SCALE = 1.0 / 64.0
L = 16


@functools.partial(
    pl.kernel,
    mesh=plsc.VectorSubcoreMesh(core_axis_name="c", subcore_axis_name="s"),
    out_type=jax.ShapeDtypeStruct((S, EMB // 8, NBT, 8, 128), jnp.float32),
    compiler_params=pltpu.CompilerParams(
        use_tc_tiling_on_sc=False, needs_layout_passes=False),
    scratch_types=(
        [pltpu.VMEM((S, BT), jnp.int32)]
        + [pltpu.VMEM((BT, EMB), jnp.float32) for _ in range(NB)]
        + [pltpu.VMEM((EMB // 8, 8, BT), jnp.float32) for _ in range(NB)]
        + [pltpu.SemaphoreType.DMA for _ in range(2 * NB)]
    ),
)
def _emb_lookup(idx_hbm, table_hbm, out_hbm, idxv,
                b0, b1, b2, b3, t0, t1, t2, t3,
                g0, g1, g2, g3, o0, o1, o2, o3):
    bufs = (b0, b1, b2, b3)
    ots = (t0, t1, t2, t3)
    gsems = (g0, g1, g2, g3)
    osems = (o0, o1, o2, o3)
    wid = lax.axis_index("s") * NC + lax.axis_index("c")
    boff = wid * BT
    lanes = lax.iota(jnp.int32, L)
    zrow = lanes * 0
    ehi = [lax.shift_right_logical(t * L + lanes, jnp.int32(3))
           for t in range(EMB // L)]
    elo = [lax.bitwise_and(t * L + lanes, jnp.int32(7))
           for t in range(EMB // L)]

    pltpu.sync_copy(idx_hbm.at[pl.ds(0, S), pl.ds(boff, BT)], idxv)

    def tile_dst(s, e8):
        return out_hbm.at[s, e8, wid, pl.ds(0, 8), pl.ds(0, 128)]

    def store_tiles(s, k):
        for e8 in range(EMB // 8):
            pltpu.async_copy(ots[k].at[e8], tile_dst(s, e8), osems[k])

    def wait_store(k):
        for e8 in range(EMB // 8):
            pltpu.make_async_copy(ots[k].at[e8], tile_dst(0, e8),
                                  osems[k]).wait()

    def start_gather(s, k):
        pltpu.async_copy(table_hbm.at[idxv.at[s]], bufs[k], gsems[k])

    def wait_gather(k):
        pltpu.make_async_copy(table_hbm.at[idxv.at[0]], bufs[k], gsems[k]).wait()

    for k in range(NB):
        start_gather(k, k)

    def group_body(g, carry):
        for k in range(NB):
            s = g * NB + k
            wait_gather(k)

            def row_body(b2, c, k=k):
                b = b2 * 2
                bss = (zrow + b, zrow + (b + 1))
                loads = [bufs[k][b + r, pl.ds(t * L, L)]
                         for r in range(2) for t in range(EMB // L)]
                scaled = [v * SCALE for v in loads]
                for r in range(2):
                    for t in range(EMB // L):
                        plsc.store_scatter(
                            ots[k], [ehi[t], elo[t], bss[r]],
                            scaled[r * (EMB // L) + t])
                return c

            lax.fori_loop(0, BT // 2, row_body, 0)
            store_tiles(s, k)

        @pl.when(g + 1 < NG)
        def _prefetch():
            for k in range(NB):
                wait_store(k)
                start_gather((g + 1) * NB + k, k)

        return carry

    lax.fori_loop(0, NG, group_body, 0)
    for k in range(NB):
        wait_store(k)


def kernel(inp, table):
    idx_t = jnp.swapaxes(inp, 0, 1)
    out = _emb_lookup(idx_t, table)
    return jnp.transpose(out, (2, 4, 0, 1, 3)).reshape(B, S, EMB)
